# baseline (speedup 1.0000x reference)
import functools

import jax
import jax.numpy as jnp
from jax import lax
from jax.experimental import pallas as pl
from jax.experimental.pallas import tpu as pltpu
from jax.experimental.pallas import tpu_sc as plsc

_LANES = 16
_CP = 128


def _gather16(v, idx):
    dnums = lax.GatherDimensionNumbers(
        offset_dims=(), collapsed_slice_dims=(0,), start_index_map=(0,))
    return lax.gather(v, idx.reshape(_LANES, 1), dnums, (1,),
                      mode=lax.GatherScatterMode.PROMISE_IN_BOUNDS)


def _bcast_lane(v, lane):
    return _gather16(v, jnp.full((_LANES,), lane, dtype=jnp.int32))


def _make_sc_kernel(B, H, W, C, NW, N):
    P = B * H * W
    HW = H * W
    PPW = P // NW
    T = PPW // N
    G = N // _LANES
    CV = C // _LANES
    assert P % NW == 0 and PPW % N == 0 and C % _LANES == 0 and N % _LANES == 0
    assert PPW % HW == 0 or HW % PPW == 0
    assert T % 2 == 0 and H == W

    mesh = plsc.VectorSubcoreMesh(core_axis_name="c", subcore_axis_name="s")

    assert 128 % N == 0

    @functools.partial(
        pl.kernel,
        mesh=mesh,
        compiler_params=pltpu.CompilerParams(use_tc_tiling_on_sc=False),
        out_type=jax.ShapeDtypeStruct((P, _CP), jnp.float32),
        scratch_types=[
            [[pltpu.VMEM((N,), jnp.float32) for _ in range(2)]
             for _ in range(2)],
            [[pltpu.VMEM((N,), jnp.int32) for _ in range(4)]
             for _ in range(2)],
            [[pltpu.VMEM((N,), jnp.float32) for _ in range(4)]
             for _ in range(2)],
            [[pltpu.VMEM((N, _CP), jnp.float32) for _ in range(4)]
             for _ in range(2)],
            [pltpu.VMEM((N, _CP), jnp.float32) for _ in range(2)],
            [pltpu.SemaphoreType.DMA for _ in range(2)],
            [pltpu.SemaphoreType.DMA for _ in range(2)],
            [pltpu.SemaphoreType.DMA for _ in range(2)],
        ],
    )
    def grid_sample(table, ggrid, out,
                    gbuf, idx_v, w_v, rows_v, out_v, sem_gr, sem_g, sem_o):
        wid = lax.axis_index("s") * 2 + lax.axis_index("c")
        base_flat = (wid * PPW // HW) * HW
        pix_base = wid * PPW

        def grid_copies(t, s):
            pix0 = pix_base + t * N
            r = pix0 // 128
            c = (pix0 % 128) // N * N
            return (
                pltpu.make_async_copy(
                    ggrid.at[r, pl.ds(c, N)], gbuf[s][0], sem_gr[s]),
                pltpu.make_async_copy(
                    ggrid.at[r, pl.ds(128 + c, N)], gbuf[s][1], sem_gr[s]),
            )

        def fire_grid(t, s):
            for cp in grid_copies(t, s):
                cp.start()

        def prep_and_fire(t, s):
            for cp in grid_copies(t, s):
                cp.wait()
            ia, ib, ic, idd = idx_v[s]
            wa, wb, wc, wd = w_v[s]
            for j in range(G):
                sl = pl.ds(j * _LANES, _LANES)
                xg = gbuf[s][0][sl]
                yg = gbuf[s][1][sl]
                x = 0.5 * ((xg + 1.0) * jnp.float32(W - 1))
                y = 0.5 * ((yg + 1.0) * jnp.float32(H - 1))
                x0i = jnp.minimum(jnp.maximum(x.astype(jnp.int32), 0), W - 1)
                y0i = jnp.minimum(jnp.maximum(y.astype(jnp.int32), 0), H - 1)
                x1i = jnp.minimum(x0i + 1, W - 1)
                y1i = jnp.minimum(y0i + 1, H - 1)
                x0f = x0i.astype(jnp.float32)
                x1f = x1i.astype(jnp.float32)
                y0f = y0i.astype(jnp.float32)
                y1f = y1i.astype(jnp.float32)
                dx0 = x1f - x
                dx1 = x - x0f
                dy0 = y1f - y
                dy1 = y - y0f
                wa[sl] = dx0 * dy0
                wb[sl] = dx0 * dy1
                wc[sl] = dx1 * dy0
                wd[sl] = dx1 * dy1
                r0 = base_flat + y0i * W
                r1 = base_flat + y1i * W
                ia[sl] = r0 + x0i
                ib[sl] = r1 + x0i
                ic[sl] = r0 + x1i
                idd[sl] = r1 + x1i
            for k in range(4):
                pltpu.async_copy(table.at[idx_v[s][k]], rows_v[s][k], sem_g[s])

        def wait_gathers(s):
            for k in range(4):
                pltpu.make_async_copy(
                    table.at[idx_v[s][k]], rows_v[s][k], sem_g[s]).wait()

        def drain_store(s):
            pltpu.make_async_copy(
                out_v[s], out.at[pl.ds(pix_base, N)], sem_o[s]).wait()

        def combine_and_store(t, s):
            ra, rb, rc, rd = rows_v[s]
            wa, wb, wc, wd = w_v[s]
            ov = out_v[s]

            GP = 4
            SPL = _LANES // GP

            def group_body(g, _):
                gs = pl.ds((g // SPL) * _LANES, _LANES)
                wav = wa[gs]
                wbv = wb[gs]
                wcv = wc[gs]
                wdv = wd[gs]
                part = (g % SPL) * GP
                for l in range(GP):
                    q = (g // SPL) * _LANES + part + l
                    lane = jnp.full((_LANES,), l, dtype=jnp.int32) + part
                    wal = _gather16(wav, lane)
                    wbl = _gather16(wbv, lane)
                    wcl = _gather16(wcv, lane)
                    wdl = _gather16(wdv, lane)
                    for j in range(CV):
                        cs = pl.ds(j * _LANES, _LANES)
                        ov[q, cs] = (wal * ra[q, cs] + wbl * rb[q, cs]
                                     + wcl * rc[q, cs] + wdl * rd[q, cs])
                return _

            lax.fori_loop(0, SPL * G, group_body, None)
            pltpu.async_copy(ov, out.at[pl.ds(pix_base + t * N, N)], sem_o[s])

        fire_grid(0, 0)
        fire_grid(1, 1)
        prep_and_fire(0, 0)

        def pair_body(t2, _):
            for bb in range(2):
                t = 2 * t2 + bb

                @pl.when(t + 2 < T)
                def _fire():
                    fire_grid(t + 2, bb)

                @pl.when(t + 1 < T)
                def _prep():
                    prep_and_fire(t + 1, 1 - bb)

                wait_gathers(bb)

                @pl.when(t >= 2)
                def _drain():
                    drain_store(bb)

                combine_and_store(t, bb)
            return _

        lax.fori_loop(0, T // 2, pair_body, None)
        drain_store(0)
        drain_store(1)

    return grid_sample


def kernel(img, grid):
    B, H, W, C = img.shape
    P = B * H * W
    table = jnp.pad(img.reshape(P, C), ((0, 0), (0, _CP - C)))
    gview = (grid.reshape(B, H, W // 128, 128, 2)
             .transpose(0, 1, 2, 4, 3)
             .reshape(B * H * (W // 128), 256))
    sc = _make_sc_kernel(B, H, W, C, NW=32, N=64)
    out = sc(table, gview)
    return out[:, :C].reshape(B, H, W, C)


# speedup vs baseline: 1.4016x; 1.4016x over previous
import functools

import jax
import jax.numpy as jnp
from jax import lax
from jax.experimental import pallas as pl
from jax.experimental.pallas import tpu as pltpu
from jax.experimental.pallas import tpu_sc as plsc

_LANES = 16
_CP = 128


def _gather16(v, idx):
    dnums = lax.GatherDimensionNumbers(
        offset_dims=(), collapsed_slice_dims=(0,), start_index_map=(0,))
    return lax.gather(v, idx.reshape(_LANES, 1), dnums, (1,),
                      mode=lax.GatherScatterMode.PROMISE_IN_BOUNDS)


def _bcast_lane(v, lane):
    return _gather16(v, jnp.full((_LANES,), lane, dtype=jnp.int32))


def _make_sc_kernel(B, H, W, C, NW, N):
    P = B * H * W
    HW = H * W
    PPW = P // NW
    T = PPW // N
    G = N // _LANES
    CV = C // _LANES
    assert P % NW == 0 and PPW % N == 0 and C % _LANES == 0 and N % _LANES == 0
    assert PPW % HW == 0 or HW % PPW == 0
    assert T % 2 == 0 and H == W

    mesh = plsc.VectorSubcoreMesh(core_axis_name="c", subcore_axis_name="s")

    assert 128 % N == 0

    @functools.partial(
        pl.kernel,
        mesh=mesh,
        compiler_params=pltpu.CompilerParams(use_tc_tiling_on_sc=False),
        out_type=jax.ShapeDtypeStruct((P, _CP), jnp.float32),
        scratch_types=[
            [[pltpu.VMEM((N,), jnp.float32) for _ in range(2)]
             for _ in range(2)],
            [[pltpu.VMEM((N,), jnp.int32) for _ in range(4)]
             for _ in range(2)],
            [[pltpu.VMEM((N,), jnp.float32) for _ in range(4)]
             for _ in range(2)],
            [[pltpu.VMEM((N, _CP), jnp.float32) for _ in range(4)]
             for _ in range(2)],
            [pltpu.VMEM((N, _CP), jnp.float32) for _ in range(2)],
            [pltpu.SemaphoreType.DMA for _ in range(2)],
            [pltpu.SemaphoreType.DMA for _ in range(2)],
            [pltpu.SemaphoreType.DMA for _ in range(2)],
        ],
    )
    def grid_sample(table, ggrid, out,
                    gbuf, idx_v, w_v, rows_v, out_v, sem_gr, sem_g, sem_o):
        wid = lax.axis_index("s") * 2 + lax.axis_index("c")
        base_flat = (wid * PPW // HW) * HW
        pix_base = wid * PPW

        def grid_copies(t, s):
            pix0 = pix_base + t * N
            r = pix0 // 128
            c = (pix0 % 128) // N * N
            return (
                pltpu.make_async_copy(
                    ggrid.at[r, pl.ds(c, N)], gbuf[s][0], sem_gr[s]),
                pltpu.make_async_copy(
                    ggrid.at[r, pl.ds(128 + c, N)], gbuf[s][1], sem_gr[s]),
            )

        def fire_grid(t, s):
            for cp in grid_copies(t, s):
                cp.start()

        def prep_and_fire(t, s):
            for cp in grid_copies(t, s):
                cp.wait()
            ia, ib, ic, idd = idx_v[s]
            wa, wb, wc, wd = w_v[s]
            for j in range(G):
                sl = pl.ds(j * _LANES, _LANES)
                xg = gbuf[s][0][sl]
                yg = gbuf[s][1][sl]
                x = 0.5 * ((xg + 1.0) * jnp.float32(W - 1))
                y = 0.5 * ((yg + 1.0) * jnp.float32(H - 1))
                x0i = jnp.minimum(jnp.maximum(x.astype(jnp.int32), 0), W - 1)
                y0i = jnp.minimum(jnp.maximum(y.astype(jnp.int32), 0), H - 1)
                x1i = jnp.minimum(x0i + 1, W - 1)
                y1i = jnp.minimum(y0i + 1, H - 1)
                x0f = x0i.astype(jnp.float32)
                x1f = x1i.astype(jnp.float32)
                y0f = y0i.astype(jnp.float32)
                y1f = y1i.astype(jnp.float32)
                dx0 = x1f - x
                dx1 = x - x0f
                dy0 = y1f - y
                dy1 = y - y0f
                wa[sl] = dx0 * dy0
                wb[sl] = dx0 * dy1
                wc[sl] = dx1 * dy0
                wd[sl] = dx1 * dy1
                r0 = base_flat + y0i * W
                r1 = base_flat + y1i * W
                ia[sl] = r0 + x0i
                ib[sl] = r1 + x0i
                ic[sl] = r0 + x1i
                idd[sl] = r1 + x1i
            for k in range(4):
                pltpu.async_copy(table.at[idx_v[s][k]], rows_v[s][k], sem_g[s])

        def wait_gathers(s):
            for k in range(4):
                pltpu.make_async_copy(
                    table.at[idx_v[s][k]], rows_v[s][k], sem_g[s]).wait()

        def drain_store(s):
            pltpu.make_async_copy(
                out_v[s], out.at[pl.ds(pix_base, N)], sem_o[s]).wait()

        def combine_and_store(t, s):
            ra, rb, rc, rd = rows_v[s]
            wa, wb, wc, wd = w_v[s]
            ov = out_v[s]

            GP = 8
            SPL = _LANES // GP

            def group_body(g, _):
                gs = pl.ds((g // SPL) * _LANES, _LANES)
                wav = wa[gs]
                wbv = wb[gs]
                wcv = wc[gs]
                wdv = wd[gs]
                part = (g % SPL) * GP
                for l in range(GP):
                    q = (g // SPL) * _LANES + part + l
                    lane = jnp.full((_LANES,), l, dtype=jnp.int32) + part
                    wal = _gather16(wav, lane)
                    wbl = _gather16(wbv, lane)
                    wcl = _gather16(wcv, lane)
                    wdl = _gather16(wdv, lane)
                    for j in range(CV):
                        cs = pl.ds(j * _LANES, _LANES)
                        ov[q, cs] = (wal * ra[q, cs] + wbl * rb[q, cs]
                                     + wcl * rc[q, cs] + wdl * rd[q, cs])
                return _

            lax.fori_loop(0, SPL * G, group_body, None)
            pltpu.async_copy(ov, out.at[pl.ds(pix_base + t * N, N)], sem_o[s])

        fire_grid(0, 0)
        fire_grid(1, 1)
        prep_and_fire(0, 0)

        def pair_body(t2, _):
            for bb in range(2):
                t = 2 * t2 + bb

                @pl.when(t + 2 < T)
                def _fire():
                    fire_grid(t + 2, bb)

                @pl.when(t + 1 < T)
                def _prep():
                    prep_and_fire(t + 1, 1 - bb)

                wait_gathers(bb)

                @pl.when(t >= 2)
                def _drain():
                    drain_store(bb)

                combine_and_store(t, bb)
            return _

        lax.fori_loop(0, T // 2, pair_body, None)
        drain_store(0)
        drain_store(1)

    return grid_sample


def kernel(img, grid):
    B, H, W, C = img.shape
    P = B * H * W
    table = jnp.pad(img.reshape(P, C), ((0, 0), (0, _CP - C)))
    gview = (grid.reshape(B, H, W // 128, 128, 2)
             .transpose(0, 1, 2, 4, 3)
             .reshape(B * H * (W // 128), 256))
    sc = _make_sc_kernel(B, H, W, C, NW=32, N=64)
    out = sc(table, gview)
    return out[:, :C].reshape(B, H, W, C)
